# baseline (speedup 1.0000x reference)
import functools

import jax
import jax.numpy as jnp
from jax import lax
from jax.experimental import pallas as pl
from jax.experimental.pallas import tpu as pltpu
from jax.experimental.pallas import tpu_sc as plsc

N = 10000
E = 320000
F = 128
De = 16
H = 128
G = 100

NC = 2
NS = 16
NW = NC * NS
PAYW = 32
CG = 128
NROW = E // CG
NLOC = NROW // NW
NEXTRA = NROW - NLOC * NW
NSTRIPE = N // NS
TE = 5000

def _silu(x):
    return x * jax.nn.sigmoid(x)


G_MC = 1
G_NMAC = NLOC // G_MC
S_MC = 2
S_MCG = S_MC * CG
S_NMAC = NLOC // S_MC


def _sc_gather_body(ps_hbm, pd_hbm, xt_hbm, src2_hbm, dst2_hbm,
                    z_out, rel_out, *scr):
    ibuf = scr[0:3]
    gs = scr[3:6]
    gd = scr[6:9]
    xs = scr[9:12]
    xd = scr[12:15]
    isem = scr[15:18]
    gsem = scr[18:21]
    wsem = scr[21:24]
    c = lax.axis_index("c")
    s = lax.axis_index("s")
    wid = s * NC + c
    row0 = wid * NLOC

    def idx_fire(u, row):
        pltpu.async_copy(src2_hbm.at[pl.ds(row, G_MC)],
                         ibuf[u].at[pl.ds(0, G_MC)], isem[u])
        pltpu.async_copy(dst2_hbm.at[pl.ds(row, G_MC)],
                         ibuf[u].at[pl.ds(G_MC, G_MC)], isem[u])

    def idx_wait(u):
        for _ in range(2):
            pltpu.make_async_copy(src2_hbm.at[pl.ds(0, G_MC)],
                                  ibuf[u].at[pl.ds(0, G_MC)], isem[u]).wait()

    def g_fire(u, nrow):
        for r in range(nrow):
            pltpu.async_copy(ps_hbm.at[ibuf[u].at[r]],
                             gs[u].at[pl.ds(r * CG, CG)], gsem[u])
            pltpu.async_copy(pd_hbm.at[ibuf[u].at[G_MC + r]],
                             gd[u].at[pl.ds(r * CG, CG)], gsem[u])
            pltpu.async_copy(xt_hbm.at[ibuf[u].at[r]],
                             xs[u].at[pl.ds(r * CG, CG)], gsem[u])
            pltpu.async_copy(xt_hbm.at[ibuf[u].at[G_MC + r]],
                             xd[u].at[pl.ds(r * CG, CG)], gsem[u])

    def g_wait(u, nrow):
        for r in range(nrow):
            pltpu.make_async_copy(ps_hbm.at[pl.ds(0, CG)],
                                  gs[u].at[pl.ds(r * CG, CG)],
                                  gsem[u]).wait()
            pltpu.make_async_copy(pd_hbm.at[pl.ds(0, CG)],
                                  gd[u].at[pl.ds(r * CG, CG)],
                                  gsem[u]).wait()
            pltpu.make_async_copy(xt_hbm.at[pl.ds(0, CG)],
                                  xs[u].at[pl.ds(r * CG, CG)],
                                  gsem[u]).wait()
            pltpu.make_async_copy(xt_hbm.at[pl.ds(0, CG)],
                                  xd[u].at[pl.ds(r * CG, CG)],
                                  gsem[u]).wait()

    def compute(u, nrow):
        def crow(r, carry):
            for cb in range(F // 32):
                sl = pl.ds(cb * 32, 32)
                gs[u][r, sl] = gs[u][r, sl] + gd[u][r, sl]
            xs[u][r, :] = xs[u][r, :] - xd[u][r, :]
            return carry
        lax.fori_loop(0, nrow * CG, crow, 0, unroll=4)

    def w_fire(u, row, nrow):
        off = pl.multiple_of(row * CG, CG)
        pltpu.async_copy(gs[u].at[pl.ds(0, nrow * CG)],
                         z_out.at[pl.ds(off, nrow * CG)], wsem[u])
        pltpu.async_copy(xs[u].at[pl.ds(0, nrow * CG)],
                         rel_out.at[pl.ds(off, nrow * CG)], wsem[u])

    def w_wait(u, nrow):
        pltpu.make_async_copy(gs[u].at[pl.ds(0, nrow * CG)],
                              z_out.at[pl.ds(0, nrow * CG)], wsem[u]).wait()
        pltpu.make_async_copy(xs[u].at[pl.ds(0, nrow * CG)],
                              rel_out.at[pl.ds(0, nrow * CG)],
                              wsem[u]).wait()

    idx_fire(0, row0 + 0)
    idx_fire(1, row0 + G_MC)
    idx_wait(0)
    g_fire(0, G_MC)
    idx_wait(1)
    g_fire(1, G_MC)
    idx_fire(2, row0 + 2 * G_MC)

    def body(j, carry):
        for u in range(3):
            e = 3 * j + u
            s_nxt = (u + 2) % 3

            def steady():
                idx_wait(s_nxt)
                if u == 0:
                    @pl.when(j >= 1)
                    def _():
                        w_wait(s_nxt, G_MC)
                else:
                    w_wait(s_nxt, G_MC)
                g_fire(s_nxt, G_MC)
            if u == 0:
                steady()
            else:
                @pl.when(e + 2 <= G_NMAC - 1)
                def _():
                    steady()
                @pl.when(e + 2 > G_NMAC - 1)
                def _():
                    w_wait(s_nxt, G_MC)
            g_wait(u, G_MC)

            @pl.when(e + 3 <= G_NMAC - 1)
            def _():
                idx_fire(u, row0 + (e + 3) * G_MC)
            compute(u, G_MC)
            w_fire(u, row0 + e * G_MC, G_MC)
        return carry

    lax.fori_loop(0, G_NMAC // 3, body, 0)
    w_wait(2, G_MC)

    @pl.when(wid < NEXTRA)
    def _():
        row = NLOC * NW + wid
        pltpu.async_copy(src2_hbm.at[pl.ds(row, 1)],
                         ibuf[0].at[pl.ds(0, 1)], isem[0])
        pltpu.async_copy(dst2_hbm.at[pl.ds(row, 1)],
                         ibuf[0].at[pl.ds(G_MC, 1)], isem[0])
        for _ in range(2):
            pltpu.make_async_copy(src2_hbm.at[pl.ds(0, 1)],
                                  ibuf[0].at[pl.ds(0, 1)], isem[0]).wait()
        g_fire(0, 1)
        g_wait(0, 1)
        compute(0, 1)
        w_fire(0, row, 1)
        w_wait(0, 1)


def _sc_scatter_body(pay_hbm, dst2_hbm, zeros_hbm, out_hbm, *scr):
    ibuf = scr[0:3]
    payv = scr[3:6]
    acc_sh = scr[6]
    lsem = scr[7:10]
    ssem = scr[10:13]
    c = lax.axis_index("c")
    s = lax.axis_index("s")
    wid = s * NC + c
    stripe = pl.ds(s * NSTRIPE, NSTRIPE)
    pltpu.sync_copy(zeros_hbm.at[stripe], acc_sh.at[stripe])
    plsc.subcore_barrier()
    row0 = wid * NLOC

    def l_fire(u, row, nrow):
        off = pl.multiple_of(row * CG, CG)
        pltpu.async_copy(dst2_hbm.at[pl.ds(row, nrow)],
                         ibuf[u].at[pl.ds(0, nrow)], lsem[u])
        pltpu.async_copy(pay_hbm.at[pl.ds(off, nrow * CG)],
                         payv[u].at[pl.ds(0, nrow * CG)], lsem[u])

    def l_wait(u, nrow):
        pltpu.make_async_copy(dst2_hbm.at[pl.ds(0, nrow)],
                              ibuf[u].at[pl.ds(0, nrow)], lsem[u]).wait()
        pltpu.make_async_copy(pay_hbm.at[pl.ds(0, nrow * CG)],
                              payv[u].at[pl.ds(0, nrow * CG)],
                              lsem[u]).wait()

    def s_fire(u, nrow):
        for r in range(nrow):
            pltpu.async_copy(payv[u].at[pl.ds(r * CG, CG)],
                             acc_sh.at[ibuf[u].at[r]], ssem[u], add=True)

    def s_wait(u, nrow):
        for r in range(nrow):
            pltpu.make_async_copy(payv[u].at[pl.ds(r * CG, CG)],
                                  acc_sh.at[pl.ds(0, CG)], ssem[u]).wait()

    l_fire(0, row0 + 0, S_MC)
    l_fire(1, row0 + S_MC, S_MC)

    def body(j, carry):
        for u in range(3):
            e = 3 * j + u
            s_nxt = (u + 2) % 3
            if u == 0:
                @pl.when(j >= 1)
                def _():
                    s_wait(s_nxt, S_MC)
                l_fire(s_nxt, row0 + (e + 2) * S_MC, S_MC)
            else:
                s_wait(s_nxt, S_MC)

                @pl.when(e + 2 <= S_NMAC - 1)
                def _():
                    l_fire(s_nxt, row0 + (e + 2) * S_MC, S_MC)
            l_wait(u, S_MC)
            s_fire(u, S_MC)
        return carry

    lax.fori_loop(0, S_NMAC // 3, body, 0)
    s_wait(2, S_MC)

    @pl.when(wid < NEXTRA)
    def _():
        row = NLOC * NW + wid
        l_fire(0, row, 1)
        l_wait(0, 1)
        s_fire(0, 1)
        s_wait(0, 1)

    plsc.subcore_barrier()
    pltpu.sync_copy(acc_sh.at[stripe], out_hbm.at[c, stripe])


@functools.cache
def _sc_kernels():
    mesh = plsc.VectorSubcoreMesh(core_axis_name="c", subcore_axis_name="s",
                                  num_cores=NC, num_subcores=NS)
    params = pltpu.CompilerParams(use_tc_tiling_on_sc=False)
    gather = pl.kernel(
        _sc_gather_body,
        out_type=(jax.ShapeDtypeStruct((E, F), jnp.bfloat16),
                  jax.ShapeDtypeStruct((E, De), jnp.float32)),
        mesh=mesh,
        compiler_params=params,
        scratch_types=(
            [pltpu.VMEM((2 * G_MC, CG), jnp.int32)] * 3
            + [pltpu.VMEM((G_MC * CG, F), jnp.bfloat16)] * 6
            + [pltpu.VMEM((G_MC * CG, De), jnp.float32)] * 6
            + [pltpu.SemaphoreType.DMA] * 9
        ),
    )
    scatter = pl.kernel(
        _sc_scatter_body,
        out_type=jax.ShapeDtypeStruct((NC, N, PAYW), jnp.float32),
        mesh=mesh,
        compiler_params=params,
        scratch_types=(
            [pltpu.VMEM((S_MC, CG), jnp.int32)] * 3
            + [pltpu.VMEM((S_MCG, PAYW), jnp.float32)] * 3
            + [pltpu.VMEM_SHARED((N, PAYW), jnp.float32)]
            + [pltpu.SemaphoreType.DMA] * 6
        ),
    )
    return gather, scatter


def _sc_gather(ps, pd, x16, src2, dst2):
    return _sc_kernels()[0](ps, pd, x16, src2, dst2)


def _sc_scatter(pay, dst2, zeros):
    return _sc_kernels()[1](pay, dst2, zeros)


def _pre_body(h_ref, ws_ref, wd_ref, be1_ref, ps_ref, pd_ref):
    h = h_ref[...]
    ps_ref[...] = jnp.dot(h, ws_ref[...], preferred_element_type=jnp.float32
                          ).astype(jnp.bfloat16)
    pd_ref[...] = (jnp.dot(h, wd_ref[...], preferred_element_type=jnp.float32)
                   + be1_ref[...]).astype(jnp.bfloat16)


BN = 2000


def _tc_pre(h, ws, wd, be1):
    return pl.pallas_call(
        _pre_body,
        grid=(N // BN,),
        in_specs=[
            pl.BlockSpec((BN, F), lambda i: (i, 0)),
            pl.BlockSpec((F, H), lambda i: (0, 0)),
            pl.BlockSpec((F, H), lambda i: (0, 0)),
            pl.BlockSpec((1, H), lambda i: (0, 0)),
        ],
        out_specs=(pl.BlockSpec((BN, F), lambda i: (i, 0)),
                   pl.BlockSpec((BN, F), lambda i: (i, 0))),
        out_shape=(jax.ShapeDtypeStruct((N, F), jnp.bfloat16),
                   jax.ShapeDtypeStruct((N, F), jnp.bfloat16)),
    )(h, ws, wd, be1.reshape(1, H))


def _edge_body(z_ref, rel_ref, e_ref, we_ref, rrow_ref, w2_ref, be2_ref,
               wx_ref, bx_ref, pay_ref):
    z = z_ref[...].astype(jnp.float32)
    rel = rel_ref[...]
    e = e_ref[...][:, :De]
    d2 = jnp.sum(rel * rel, axis=1, keepdims=True)
    z = z + jnp.dot(e, we_ref[...], preferred_element_type=jnp.float32)
    z = z + d2 * rrow_ref[...]
    m = _silu(z)
    e_new = _silu(jnp.dot(m, w2_ref[...], preferred_element_type=jnp.float32)
                  + be2_ref[...])
    w = jnp.tanh(jnp.sum(e_new * wx_ref[...], axis=1, keepdims=True)
                 + bx_ref[...])
    lane = lax.broadcasted_iota(jnp.int32, (z.shape[0], De), 1)
    ones_col = jnp.where(lane == De - 1, 1.0, 0.0)
    pay_ref[...] = jnp.concatenate([e_new, rel * w + ones_col], axis=1)


def _tc_edge(z, rel, e_src, we, rrow, w2, be2, wx, bx):
    ecols = e_src.shape[1]
    return pl.pallas_call(
        _edge_body,
        grid=(E // TE,),
        in_specs=[
            pl.BlockSpec((TE, F), lambda i: (i, 0)),
            pl.BlockSpec((TE, De), lambda i: (i, 0)),
            pl.BlockSpec((TE, ecols), lambda i: (i, 0)),
            pl.BlockSpec((De, H), lambda i: (0, 0)),
            pl.BlockSpec((1, H), lambda i: (0, 0)),
            pl.BlockSpec((H, De), lambda i: (0, 0)),
            pl.BlockSpec((1, De), lambda i: (0, 0)),
            pl.BlockSpec((1, De), lambda i: (0, 0)),
            pl.BlockSpec((1, 1), lambda i: (0, 0)),
        ],
        out_specs=pl.BlockSpec((TE, PAYW), lambda i: (i, 0)),
        out_shape=jax.ShapeDtypeStruct((E, PAYW), jnp.float32),
    )(z, rel, e_src, we, rrow.reshape(1, H), w2, be2.reshape(1, De),
      wx.reshape(1, De), bx.reshape(1, 1))


def _node_body(h_ref, x16_ref, acc_ref, w1a_ref, w1b_ref, bh1_ref,
               w2_ref, bh2_ref, *out_refs, has_next):
    h = h_ref[...]
    x16 = x16_ref[...]
    a = acc_ref[0] + acc_ref[1]
    deg = jnp.maximum(a[:, PAYW - 1:PAYW], 1.0)
    agg = a[:, :De] / deg
    lane = lax.broadcasted_iota(jnp.int32, (h.shape[0], De), 1)
    xmask = jnp.where(lane < 3, 1.0, 0.0)
    x16_new = x16 + (a[:, De:PAYW] * xmask) / deg
    t = _silu(jnp.dot(h, w1a_ref[...], preferred_element_type=jnp.float32)
              + jnp.dot(agg, w1b_ref[...], preferred_element_type=jnp.float32)
              + bh1_ref[...])
    h_new = h + jnp.dot(t, w2_ref[...], preferred_element_type=jnp.float32) \
        + bh2_ref[...]
    if has_next:
        hn_ref, xn_ref, ps_ref, pd_ref, wsn_ref, wdn_ref, be1n_ref = out_refs
        ps_ref[...] = jnp.dot(h_new, wsn_ref[...],
                              preferred_element_type=jnp.float32
                              ).astype(jnp.bfloat16)
        pd_ref[...] = (jnp.dot(h_new, wdn_ref[...],
                               preferred_element_type=jnp.float32)
                       + be1n_ref[...]).astype(jnp.bfloat16)
    else:
        hn_ref, xn_ref = out_refs
    hn_ref[...] = h_new
    xn_ref[...] = x16_new


def _tc_node(h, x16, acc, w1a, w1b, bh1, w2, bh2, nxt):
    base_in = [
        pl.BlockSpec((BN, F), lambda i: (i, 0)),
        pl.BlockSpec((BN, De), lambda i: (i, 0)),
        pl.BlockSpec((NC, BN, PAYW), lambda i: (0, i, 0)),
        pl.BlockSpec((F, H), lambda i: (0, 0)),
        pl.BlockSpec((De, H), lambda i: (0, 0)),
        pl.BlockSpec((1, H), lambda i: (0, 0)),
        pl.BlockSpec((H, F), lambda i: (0, 0)),
        pl.BlockSpec((1, F), lambda i: (0, 0)),
    ]
    base_out = [pl.BlockSpec((BN, F), lambda i: (i, 0)),
                pl.BlockSpec((BN, De), lambda i: (i, 0))]
    if nxt is None:
        def body(h_ref, x16_ref, acc_ref, w1a_ref, w1b_ref, bh1_ref,
                 w2_ref, bh2_ref, hn_ref, xn_ref):
            _node_body(h_ref, x16_ref, acc_ref, w1a_ref, w1b_ref, bh1_ref,
                       w2_ref, bh2_ref, hn_ref, xn_ref, has_next=False)
        return pl.pallas_call(
            body,
            grid=(N // BN,),
            in_specs=base_in,
            out_specs=tuple(base_out),
            out_shape=(jax.ShapeDtypeStruct((N, F), jnp.float32),
                       jax.ShapeDtypeStruct((N, De), jnp.float32)),
        )(h, x16, acc, w1a, w1b, bh1.reshape(1, H), w2, bh2.reshape(1, F))
    wsn, wdn, be1n = nxt

    def body(h_ref, x16_ref, acc_ref, w1a_ref, w1b_ref, bh1_ref,
             w2_ref, bh2_ref, wsn_ref, wdn_ref, be1n_ref,
             hn_ref, xn_ref, ps_ref, pd_ref):
        _node_body(h_ref, x16_ref, acc_ref, w1a_ref, w1b_ref, bh1_ref,
                   w2_ref, bh2_ref, hn_ref, xn_ref, ps_ref, pd_ref,
                   wsn_ref, wdn_ref, be1n_ref, has_next=True)
    return pl.pallas_call(
        body,
        grid=(N // BN,),
        in_specs=base_in + [
            pl.BlockSpec((F, H), lambda i: (0, 0)),
            pl.BlockSpec((F, H), lambda i: (0, 0)),
            pl.BlockSpec((1, H), lambda i: (0, 0)),
        ],
        out_specs=tuple(base_out) + (pl.BlockSpec((BN, F), lambda i: (i, 0)),
                                     pl.BlockSpec((BN, F), lambda i: (i, 0))),
        out_shape=(jax.ShapeDtypeStruct((N, F), jnp.float32),
                   jax.ShapeDtypeStruct((N, De), jnp.float32),
                   jax.ShapeDtypeStruct((N, F), jnp.bfloat16),
                   jax.ShapeDtypeStruct((N, F), jnp.bfloat16)),
    )(h, x16, acc, w1a, w1b, bh1.reshape(1, H), w2, bh2.reshape(1, F),
      wsn, wdn, be1n.reshape(1, H))


NBLK = N // BN


def _mono_a_body(h_ref, nfc_ref, gid_ref, wm_ref, bm_ref,
                 pm_ref, sums_ref, counts_ref):
    h = h_ref[...]
    pm = jnp.sum(h * wm_ref[...], axis=1, keepdims=True) + bm_ref[...]
    pm = jnp.where(nfc_ref[...] == 1.0, jnp.abs(pm), pm)
    pm_ref[...] = pm
    gcol = lax.broadcasted_iota(jnp.int32, (BN, G), 1)
    onehot = jnp.where(gid_ref[...] == gcol, 1.0, 0.0)
    sums_ref[...] = lax.dot_general(
        pm, onehot, (((0,), (0,)), ((), ())),
        preferred_element_type=jnp.float32).reshape(1, 1, G)
    counts_ref[...] = jnp.sum(onehot, axis=0).reshape(1, 1, G)


def _mono_b_body(pm_ref, gid_ref, sums_ref, counts_ref, out_ref):
    sums = jnp.sum(sums_ref[...], axis=0)
    counts = jnp.maximum(jnp.sum(counts_ref[...], axis=0), 1.0)
    fv = sums / counts
    fv = jnp.where(jnp.abs(sums) < 0.01, 0.0, fv)
    gcol = lax.broadcasted_iota(jnp.int32, (BN, G), 1)
    onehot = jnp.where(gid_ref[...] == gcol, 1.0, 0.0)
    corr = lax.dot_general(onehot, fv, (((1,), (1,)), ((), ())),
                           preferred_element_type=jnp.float32)
    out_ref[...] = pm_ref[...] - corr


def _tc_mono(h, nfc, gid, wm, bm):
    gid2 = gid.reshape(N, 1)
    pm1, sums, counts = pl.pallas_call(
        _mono_a_body,
        grid=(NBLK,),
        in_specs=[
            pl.BlockSpec((BN, F), lambda i: (i, 0)),
            pl.BlockSpec((BN, 1), lambda i: (i, 0)),
            pl.BlockSpec((BN, 1), lambda i: (i, 0)),
            pl.BlockSpec((1, H), lambda i: (0, 0)),
            pl.BlockSpec((1, 1), lambda i: (0, 0)),
        ],
        out_specs=(pl.BlockSpec((BN, 1), lambda i: (i, 0)),
                   pl.BlockSpec((1, 1, G), lambda i: (i, 0, 0)),
                   pl.BlockSpec((1, 1, G), lambda i: (i, 0, 0))),
        out_shape=(jax.ShapeDtypeStruct((N, 1), jnp.float32),
                   jax.ShapeDtypeStruct((NBLK, 1, G), jnp.float32),
                   jax.ShapeDtypeStruct((NBLK, 1, G), jnp.float32)),
    )(h, nfc, gid2, wm.reshape(1, H), bm.reshape(1, 1))
    return pl.pallas_call(
        _mono_b_body,
        grid=(NBLK,),
        in_specs=[
            pl.BlockSpec((BN, 1), lambda i: (i, 0)),
            pl.BlockSpec((BN, 1), lambda i: (i, 0)),
            pl.BlockSpec((NBLK, 1, G), lambda i: (0, 0, 0)),
            pl.BlockSpec((NBLK, 1, G), lambda i: (0, 0, 0)),
        ],
        out_specs=pl.BlockSpec((BN, 1), lambda i: (i, 0)),
        out_shape=jax.ShapeDtypeStruct((N, 1), jnp.float32),
    )(pm1, gid2, sums, counts)


def _colmask(width, cols):
    lane = lax.broadcasted_iota(jnp.int32, (1, width), 1)
    m = jnp.zeros((1, width), jnp.float32)
    for ci in cols:
        m = jnp.where(lane == ci, 1.0, m)
    return m


def _final_body(pm_ref, hd_ref, hq_ref, ho_ref, wd_ref, bd_ref,
                wq_ref, bq_ref, wo_ref, bo_ref, out_ref):
    pd = jnp.dot(hd_ref[...], wd_ref[...],
                 preferred_element_type=jnp.float32) + bd_ref[...]
    pq = jnp.dot(hq_ref[...], wq_ref[...],
                 preferred_element_type=jnp.float32) + bq_ref[...]
    mt = (pq[:, 0:1] + pq[:, 3:4] + pq[:, 5:6]) / 3.0
    pq = pq - mt * _colmask(6, (0, 3, 5))
    po = jnp.dot(ho_ref[...], wo_ref[...],
                 preferred_element_type=jnp.float32) + bo_ref[...]
    mt0 = (po[:, 0:1] + po[:, 3:4] + po[:, 5:6]) / 3.0
    mt1 = (po[:, 6:7] + po[:, 1:2] + po[:, 8:9]) / 3.0
    mt2 = (po[:, 9:10] + po[:, 2:3] + po[:, 7:8]) / 3.0
    po = po - mt0 * _colmask(10, (0, 3, 5)) \
            - mt1 * _colmask(10, (6, 1, 8)) \
            - mt2 * _colmask(10, (9, 2, 7))
    out_ref[...] = jnp.concatenate([pm_ref[...], pd, pq, po], axis=1)


def _tc_final(pm, hd, hq, ho, wd, bd, wq, bq, wo, bo):
    return pl.pallas_call(
        _final_body,
        grid=(NBLK,),
        in_specs=[
            pl.BlockSpec((BN, 1), lambda i: (i, 0)),
            pl.BlockSpec((BN, F), lambda i: (i, 0)),
            pl.BlockSpec((BN, F), lambda i: (i, 0)),
            pl.BlockSpec((BN, F), lambda i: (i, 0)),
            pl.BlockSpec((F, 3), lambda i: (0, 0)),
            pl.BlockSpec((1, 3), lambda i: (0, 0)),
            pl.BlockSpec((F, 6), lambda i: (0, 0)),
            pl.BlockSpec((1, 6), lambda i: (0, 0)),
            pl.BlockSpec((F, 10), lambda i: (0, 0)),
            pl.BlockSpec((1, 10), lambda i: (0, 0)),
        ],
        out_specs=pl.BlockSpec((BN, 20), lambda i: (i, 0)),
        out_shape=jax.ShapeDtypeStruct((N, 20), jnp.float32),
    )(pm, hd, hq, ho, wd, bd.reshape(1, 3), wq, bq.reshape(1, 6),
      wo, bo.reshape(1, 10))


def _branches(nfeats, x16_0, efeats, src2, dst2, zeros_acc,
              We1, be1, We2, be2, Wx, bx, Wh1, bh1, Wh2, bh2):
    NB = 4
    st = []
    for b in range(NB):
        i0 = b * 5
        ps, pd = _tc_pre(nfeats, We1[i0, :F], We1[i0, F:2 * F], be1[i0])
        st.append([nfeats, x16_0, efeats, ps, pd])
    for l in range(5):
        zr = [_sc_gather(st[b][3], st[b][4], st[b][1], src2, dst2)
              for b in range(NB)]
        pays = [_tc_edge(zr[b][0], zr[b][1], st[b][2],
                         We1[i, 2 * F:2 * F + De], We1[i, 2 * F + De],
                         We2[i], be2[i], Wx[i, :, 0], bx[i])
                for b in range(NB) for i in [b * 5 + l]]
        accs = [_sc_scatter(pays[b], dst2, zeros_acc) for b in range(NB)]
        for b in range(NB):
            i = b * 5 + l
            nxt = None if l == 4 else (We1[i + 1, :F], We1[i + 1, F:2 * F],
                                       be1[i + 1])
            outs = _tc_node(st[b][0], st[b][1], accs[b], Wh1[i, :F],
                            Wh1[i, F:], bh1[i], Wh2[i], bh2[i], nxt)
            if l == 4:
                st[b] = [outs[0], outs[1], pays[b], None, None]
            else:
                st[b] = [outs[0], outs[1], pays[b], outs[2], outs[3]]
    return [st[b][0] for b in range(4)]


def kernel(nfeats, coordinates, efeats, edge_index, node_graph_ids,
           We1, be1, We2, be2, Wx, bx, Wh1, bh1, Wh2, bh2,
           Wm, bm, Wd, bd, Wq, bq, Wo, bo):
    src2 = edge_index[0].reshape(NROW, CG)
    dst2 = edge_index[1].reshape(NROW, CG)
    x16_0 = jnp.pad(coordinates, ((0, 0), (0, De - 3)))
    zeros_acc = jnp.zeros((N, PAYW), jnp.float32)
    h_mon, h_dip, h_quad, h_oct = _branches(
        nfeats, x16_0, efeats, src2, dst2, zeros_acc,
        We1, be1, We2, be2, Wx, bx, Wh1, bh1, Wh2, bh2)
    pm = _tc_mono(h_mon, nfeats[:, 0:1], node_graph_ids, Wm[:, 0], bm)
    return _tc_final(pm, h_dip, h_quad, h_oct, Wd, bd, Wq, bq, Wo, bo)


# speedup vs baseline: 1.3406x; 1.3406x over previous
import functools

import jax
import jax.numpy as jnp
from jax import lax
from jax.experimental import pallas as pl
from jax.experimental.pallas import tpu as pltpu
from jax.experimental.pallas import tpu_sc as plsc

N = 10000
E = 320000
F = 128
De = 16
H = 128
G = 100

NC = 2
NS = 16
NW = NC * NS
PAYW = 32
CG = 128
NROW = E // CG
NLOC = NROW // NW
NEXTRA = NROW - NLOC * NW
NSTRIPE = N // NS
TE = 5000

def _silu(x):
    return x * jax.nn.sigmoid(x)


G_MC = 1
G_NMAC = NLOC // G_MC
S_MC = 2
S_MCG = S_MC * CG
S_NMAC = NLOC // S_MC


def _sc_gather_body(ps_hbm, pd_hbm, xt_hbm, src2_hbm, dst2_hbm,
                    z_out, rel_out, *scr):
    ibuf = scr[0:3]
    gs = scr[3:6]
    gd = scr[6:9]
    xs = scr[9:12]
    xd = scr[12:15]
    isem = scr[15:18]
    gsem = scr[18:21]
    wsem = scr[21:24]
    c = lax.axis_index("c")
    s = lax.axis_index("s")
    wid = s * NC + c
    row0 = wid * NLOC

    def idx_fire(u, row):
        pltpu.async_copy(src2_hbm.at[pl.ds(row, G_MC)],
                         ibuf[u].at[pl.ds(0, G_MC)], isem[u])
        pltpu.async_copy(dst2_hbm.at[pl.ds(row, G_MC)],
                         ibuf[u].at[pl.ds(G_MC, G_MC)], isem[u])

    def idx_wait(u):
        for _ in range(2):
            pltpu.make_async_copy(src2_hbm.at[pl.ds(0, G_MC)],
                                  ibuf[u].at[pl.ds(0, G_MC)], isem[u]).wait()

    def g_fire(u, nrow):
        for r in range(nrow):
            pltpu.async_copy(ps_hbm.at[ibuf[u].at[r]],
                             gs[u].at[pl.ds(r * CG, CG)], gsem[u])
            pltpu.async_copy(pd_hbm.at[ibuf[u].at[G_MC + r]],
                             gd[u].at[pl.ds(r * CG, CG)], gsem[u])
            pltpu.async_copy(xt_hbm.at[ibuf[u].at[r]],
                             xs[u].at[pl.ds(r * CG, CG)], gsem[u])
            pltpu.async_copy(xt_hbm.at[ibuf[u].at[G_MC + r]],
                             xd[u].at[pl.ds(r * CG, CG)], gsem[u])

    def g_wait(u, nrow):
        for r in range(nrow):
            pltpu.make_async_copy(ps_hbm.at[pl.ds(0, CG)],
                                  gs[u].at[pl.ds(r * CG, CG)],
                                  gsem[u]).wait()
            pltpu.make_async_copy(pd_hbm.at[pl.ds(0, CG)],
                                  gd[u].at[pl.ds(r * CG, CG)],
                                  gsem[u]).wait()
            pltpu.make_async_copy(xt_hbm.at[pl.ds(0, CG)],
                                  xs[u].at[pl.ds(r * CG, CG)],
                                  gsem[u]).wait()
            pltpu.make_async_copy(xt_hbm.at[pl.ds(0, CG)],
                                  xd[u].at[pl.ds(r * CG, CG)],
                                  gsem[u]).wait()

    def compute(u, nrow):
        def crow(r, carry):
            for cb in range(F // 16):
                sl = pl.ds(cb * 16, 16)
                gs[u][r, sl] = gs[u][r, sl] + gd[u][r, sl]
            xs[u][r, :] = xs[u][r, :] - xd[u][r, :]
            return carry
        lax.fori_loop(0, nrow * CG, crow, 0, unroll=4)

    def w_fire(u, row, nrow):
        off = pl.multiple_of(row * CG, CG)
        pltpu.async_copy(gs[u].at[pl.ds(0, nrow * CG)],
                         z_out.at[pl.ds(off, nrow * CG)], wsem[u])
        pltpu.async_copy(xs[u].at[pl.ds(0, nrow * CG)],
                         rel_out.at[pl.ds(off, nrow * CG)], wsem[u])

    def w_wait(u, nrow):
        pltpu.make_async_copy(gs[u].at[pl.ds(0, nrow * CG)],
                              z_out.at[pl.ds(0, nrow * CG)], wsem[u]).wait()
        pltpu.make_async_copy(xs[u].at[pl.ds(0, nrow * CG)],
                              rel_out.at[pl.ds(0, nrow * CG)],
                              wsem[u]).wait()

    idx_fire(0, row0 + 0)
    idx_fire(1, row0 + G_MC)
    idx_wait(0)
    g_fire(0, G_MC)
    idx_wait(1)
    g_fire(1, G_MC)
    idx_fire(2, row0 + 2 * G_MC)

    def body(j, carry):
        for u in range(3):
            e = 3 * j + u
            s_nxt = (u + 2) % 3

            def steady():
                idx_wait(s_nxt)
                if u == 0:
                    @pl.when(j >= 1)
                    def _():
                        w_wait(s_nxt, G_MC)
                else:
                    w_wait(s_nxt, G_MC)
                g_fire(s_nxt, G_MC)
            if u == 0:
                steady()
            else:
                @pl.when(e + 2 <= G_NMAC - 1)
                def _():
                    steady()
                @pl.when(e + 2 > G_NMAC - 1)
                def _():
                    w_wait(s_nxt, G_MC)
            g_wait(u, G_MC)

            @pl.when(e + 3 <= G_NMAC - 1)
            def _():
                idx_fire(u, row0 + (e + 3) * G_MC)
            compute(u, G_MC)
            w_fire(u, row0 + e * G_MC, G_MC)
        return carry

    lax.fori_loop(0, G_NMAC // 3, body, 0)
    w_wait(2, G_MC)

    @pl.when(wid < NEXTRA)
    def _():
        row = NLOC * NW + wid
        pltpu.async_copy(src2_hbm.at[pl.ds(row, 1)],
                         ibuf[0].at[pl.ds(0, 1)], isem[0])
        pltpu.async_copy(dst2_hbm.at[pl.ds(row, 1)],
                         ibuf[0].at[pl.ds(G_MC, 1)], isem[0])
        for _ in range(2):
            pltpu.make_async_copy(src2_hbm.at[pl.ds(0, 1)],
                                  ibuf[0].at[pl.ds(0, 1)], isem[0]).wait()
        g_fire(0, 1)
        g_wait(0, 1)
        compute(0, 1)
        w_fire(0, row, 1)
        w_wait(0, 1)


def _sc_scatter_body(pay_hbm, dst2_hbm, zeros_hbm, out_hbm, *scr):
    ibuf = scr[0:3]
    payv = scr[3:6]
    acc_sh = scr[6]
    lsem = scr[7:10]
    ssem = scr[10:13]
    c = lax.axis_index("c")
    s = lax.axis_index("s")
    wid = s * NC + c
    stripe = pl.ds(s * NSTRIPE, NSTRIPE)
    pltpu.sync_copy(zeros_hbm.at[stripe], acc_sh.at[stripe])
    plsc.subcore_barrier()
    row0 = wid * NLOC

    def l_fire(u, row, nrow):
        off = pl.multiple_of(row * CG, CG)
        pltpu.async_copy(dst2_hbm.at[pl.ds(row, nrow)],
                         ibuf[u].at[pl.ds(0, nrow)], lsem[u])
        pltpu.async_copy(pay_hbm.at[pl.ds(off, nrow * CG)],
                         payv[u].at[pl.ds(0, nrow * CG)], lsem[u])

    def l_wait(u, nrow):
        pltpu.make_async_copy(dst2_hbm.at[pl.ds(0, nrow)],
                              ibuf[u].at[pl.ds(0, nrow)], lsem[u]).wait()
        pltpu.make_async_copy(pay_hbm.at[pl.ds(0, nrow * CG)],
                              payv[u].at[pl.ds(0, nrow * CG)],
                              lsem[u]).wait()

    def s_fire(u, nrow):
        for r in range(nrow):
            pltpu.async_copy(payv[u].at[pl.ds(r * CG, CG)],
                             acc_sh.at[ibuf[u].at[r]], ssem[u], add=True)

    def s_wait(u, nrow):
        for r in range(nrow):
            pltpu.make_async_copy(payv[u].at[pl.ds(r * CG, CG)],
                                  acc_sh.at[pl.ds(0, CG)], ssem[u]).wait()

    l_fire(0, row0 + 0, S_MC)
    l_fire(1, row0 + S_MC, S_MC)

    def body(j, carry):
        for u in range(3):
            e = 3 * j + u
            s_nxt = (u + 2) % 3
            if u == 0:
                @pl.when(j >= 1)
                def _():
                    s_wait(s_nxt, S_MC)
                l_fire(s_nxt, row0 + (e + 2) * S_MC, S_MC)
            else:
                s_wait(s_nxt, S_MC)

                @pl.when(e + 2 <= S_NMAC - 1)
                def _():
                    l_fire(s_nxt, row0 + (e + 2) * S_MC, S_MC)
            l_wait(u, S_MC)
            s_fire(u, S_MC)
        return carry

    lax.fori_loop(0, S_NMAC // 3, body, 0)
    s_wait(2, S_MC)

    @pl.when(wid < NEXTRA)
    def _():
        row = NLOC * NW + wid
        l_fire(0, row, 1)
        l_wait(0, 1)
        s_fire(0, 1)
        s_wait(0, 1)

    plsc.subcore_barrier()
    pltpu.sync_copy(acc_sh.at[stripe], out_hbm.at[c, stripe])


@functools.cache
def _sc_kernels():
    mesh = plsc.VectorSubcoreMesh(core_axis_name="c", subcore_axis_name="s",
                                  num_cores=NC, num_subcores=NS)
    params = pltpu.CompilerParams(use_tc_tiling_on_sc=False)
    gather = pl.kernel(
        _sc_gather_body,
        out_type=(jax.ShapeDtypeStruct((E, F), jnp.float32),
                  jax.ShapeDtypeStruct((E, De), jnp.float32)),
        mesh=mesh,
        compiler_params=params,
        scratch_types=(
            [pltpu.VMEM((2 * G_MC, CG), jnp.int32)] * 3
            + [pltpu.VMEM((G_MC * CG, F), jnp.float32)] * 6
            + [pltpu.VMEM((G_MC * CG, De), jnp.float32)] * 6
            + [pltpu.SemaphoreType.DMA] * 9
        ),
    )
    scatter = pl.kernel(
        _sc_scatter_body,
        out_type=jax.ShapeDtypeStruct((NC, N, PAYW), jnp.float32),
        mesh=mesh,
        compiler_params=params,
        scratch_types=(
            [pltpu.VMEM((S_MC, CG), jnp.int32)] * 3
            + [pltpu.VMEM((S_MCG, PAYW), jnp.float32)] * 3
            + [pltpu.VMEM_SHARED((N, PAYW), jnp.float32)]
            + [pltpu.SemaphoreType.DMA] * 6
        ),
    )
    return gather, scatter


def _sc_gather(ps, pd, x16, src2, dst2):
    return _sc_kernels()[0](ps, pd, x16, src2, dst2)


def _sc_scatter(pay, dst2, zeros):
    return _sc_kernels()[1](pay, dst2, zeros)


def _pre_body(h_ref, ws_ref, wd_ref, be1_ref, ps_ref, pd_ref):
    h = h_ref[...]
    ps_ref[...] = jnp.dot(h, ws_ref[...], preferred_element_type=jnp.float32)
    pd_ref[...] = jnp.dot(h, wd_ref[...],
                          preferred_element_type=jnp.float32) + be1_ref[...]


BN = 2000


def _tc_pre(h, ws, wd, be1):
    return pl.pallas_call(
        _pre_body,
        grid=(N // BN,),
        in_specs=[
            pl.BlockSpec((BN, F), lambda i: (i, 0)),
            pl.BlockSpec((F, H), lambda i: (0, 0)),
            pl.BlockSpec((F, H), lambda i: (0, 0)),
            pl.BlockSpec((1, H), lambda i: (0, 0)),
        ],
        out_specs=(pl.BlockSpec((BN, F), lambda i: (i, 0)),
                   pl.BlockSpec((BN, F), lambda i: (i, 0))),
        out_shape=(jax.ShapeDtypeStruct((N, F), jnp.float32),
                   jax.ShapeDtypeStruct((N, F), jnp.float32)),
    )(h, ws, wd, be1.reshape(1, H))


def _edge_body(z_ref, rel_ref, e_ref, we_ref, rrow_ref, w2_ref, be2_ref,
               wx_ref, bx_ref, pay_ref):
    z = z_ref[...].astype(jnp.float32)
    rel = rel_ref[...]
    e = e_ref[...][:, :De]
    d2 = jnp.sum(rel * rel, axis=1, keepdims=True)
    z = z + jnp.dot(e, we_ref[...], preferred_element_type=jnp.float32)
    z = z + d2 * rrow_ref[...]
    m = _silu(z)
    e_new = _silu(jnp.dot(m, w2_ref[...], preferred_element_type=jnp.float32)
                  + be2_ref[...])
    w = jnp.tanh(jnp.sum(e_new * wx_ref[...], axis=1, keepdims=True)
                 + bx_ref[...])
    lane = lax.broadcasted_iota(jnp.int32, (z.shape[0], De), 1)
    ones_col = jnp.where(lane == De - 1, 1.0, 0.0)
    pay_ref[...] = jnp.concatenate([e_new, rel * w + ones_col], axis=1)


def _tc_edge(z, rel, e_src, we, rrow, w2, be2, wx, bx):
    ecols = e_src.shape[1]
    return pl.pallas_call(
        _edge_body,
        grid=(E // TE,),
        in_specs=[
            pl.BlockSpec((TE, F), lambda i: (i, 0)),
            pl.BlockSpec((TE, De), lambda i: (i, 0)),
            pl.BlockSpec((TE, ecols), lambda i: (i, 0)),
            pl.BlockSpec((De, H), lambda i: (0, 0)),
            pl.BlockSpec((1, H), lambda i: (0, 0)),
            pl.BlockSpec((H, De), lambda i: (0, 0)),
            pl.BlockSpec((1, De), lambda i: (0, 0)),
            pl.BlockSpec((1, De), lambda i: (0, 0)),
            pl.BlockSpec((1, 1), lambda i: (0, 0)),
        ],
        out_specs=pl.BlockSpec((TE, PAYW), lambda i: (i, 0)),
        out_shape=jax.ShapeDtypeStruct((E, PAYW), jnp.float32),
    )(z, rel, e_src, we, rrow.reshape(1, H), w2, be2.reshape(1, De),
      wx.reshape(1, De), bx.reshape(1, 1))


def _node_body(h_ref, x16_ref, acc_ref, w1a_ref, w1b_ref, bh1_ref,
               w2_ref, bh2_ref, *out_refs, has_next):
    h = h_ref[...]
    x16 = x16_ref[...]
    a = acc_ref[0] + acc_ref[1]
    deg = jnp.maximum(a[:, PAYW - 1:PAYW], 1.0)
    agg = a[:, :De] / deg
    lane = lax.broadcasted_iota(jnp.int32, (h.shape[0], De), 1)
    xmask = jnp.where(lane < 3, 1.0, 0.0)
    x16_new = x16 + (a[:, De:PAYW] * xmask) / deg
    t = _silu(jnp.dot(h, w1a_ref[...], preferred_element_type=jnp.float32)
              + jnp.dot(agg, w1b_ref[...], preferred_element_type=jnp.float32)
              + bh1_ref[...])
    h_new = h + jnp.dot(t, w2_ref[...], preferred_element_type=jnp.float32) \
        + bh2_ref[...]
    if has_next:
        hn_ref, xn_ref, ps_ref, pd_ref, wsn_ref, wdn_ref, be1n_ref = out_refs
        ps_ref[...] = jnp.dot(h_new, wsn_ref[...],
                              preferred_element_type=jnp.float32)
        pd_ref[...] = jnp.dot(h_new, wdn_ref[...],
                              preferred_element_type=jnp.float32) \
            + be1n_ref[...]
    else:
        hn_ref, xn_ref = out_refs
    hn_ref[...] = h_new
    xn_ref[...] = x16_new


def _tc_node(h, x16, acc, w1a, w1b, bh1, w2, bh2, nxt):
    base_in = [
        pl.BlockSpec((BN, F), lambda i: (i, 0)),
        pl.BlockSpec((BN, De), lambda i: (i, 0)),
        pl.BlockSpec((NC, BN, PAYW), lambda i: (0, i, 0)),
        pl.BlockSpec((F, H), lambda i: (0, 0)),
        pl.BlockSpec((De, H), lambda i: (0, 0)),
        pl.BlockSpec((1, H), lambda i: (0, 0)),
        pl.BlockSpec((H, F), lambda i: (0, 0)),
        pl.BlockSpec((1, F), lambda i: (0, 0)),
    ]
    base_out = [pl.BlockSpec((BN, F), lambda i: (i, 0)),
                pl.BlockSpec((BN, De), lambda i: (i, 0))]
    if nxt is None:
        def body(h_ref, x16_ref, acc_ref, w1a_ref, w1b_ref, bh1_ref,
                 w2_ref, bh2_ref, hn_ref, xn_ref):
            _node_body(h_ref, x16_ref, acc_ref, w1a_ref, w1b_ref, bh1_ref,
                       w2_ref, bh2_ref, hn_ref, xn_ref, has_next=False)
        return pl.pallas_call(
            body,
            grid=(N // BN,),
            in_specs=base_in,
            out_specs=tuple(base_out),
            out_shape=(jax.ShapeDtypeStruct((N, F), jnp.float32),
                       jax.ShapeDtypeStruct((N, De), jnp.float32)),
        )(h, x16, acc, w1a, w1b, bh1.reshape(1, H), w2, bh2.reshape(1, F))
    wsn, wdn, be1n = nxt

    def body(h_ref, x16_ref, acc_ref, w1a_ref, w1b_ref, bh1_ref,
             w2_ref, bh2_ref, wsn_ref, wdn_ref, be1n_ref,
             hn_ref, xn_ref, ps_ref, pd_ref):
        _node_body(h_ref, x16_ref, acc_ref, w1a_ref, w1b_ref, bh1_ref,
                   w2_ref, bh2_ref, hn_ref, xn_ref, ps_ref, pd_ref,
                   wsn_ref, wdn_ref, be1n_ref, has_next=True)
    return pl.pallas_call(
        body,
        grid=(N // BN,),
        in_specs=base_in + [
            pl.BlockSpec((F, H), lambda i: (0, 0)),
            pl.BlockSpec((F, H), lambda i: (0, 0)),
            pl.BlockSpec((1, H), lambda i: (0, 0)),
        ],
        out_specs=tuple(base_out) + (pl.BlockSpec((BN, F), lambda i: (i, 0)),
                                     pl.BlockSpec((BN, F), lambda i: (i, 0))),
        out_shape=(jax.ShapeDtypeStruct((N, F), jnp.float32),
                   jax.ShapeDtypeStruct((N, De), jnp.float32),
                   jax.ShapeDtypeStruct((N, F), jnp.float32),
                   jax.ShapeDtypeStruct((N, F), jnp.float32)),
    )(h, x16, acc, w1a, w1b, bh1.reshape(1, H), w2, bh2.reshape(1, F),
      wsn, wdn, be1n.reshape(1, H))


NBLK = N // BN


def _mono_a_body(h_ref, nfc_ref, gid_ref, wm_ref, bm_ref,
                 pm_ref, sums_ref, counts_ref):
    h = h_ref[...]
    pm = jnp.sum(h * wm_ref[...], axis=1, keepdims=True) + bm_ref[...]
    pm = jnp.where(nfc_ref[...] == 1.0, jnp.abs(pm), pm)
    pm_ref[...] = pm
    gcol = lax.broadcasted_iota(jnp.int32, (BN, G), 1)
    onehot = jnp.where(gid_ref[...] == gcol, 1.0, 0.0)
    sums_ref[...] = lax.dot_general(
        pm, onehot, (((0,), (0,)), ((), ())),
        preferred_element_type=jnp.float32).reshape(1, 1, G)
    counts_ref[...] = jnp.sum(onehot, axis=0).reshape(1, 1, G)


def _mono_b_body(pm_ref, gid_ref, sums_ref, counts_ref, out_ref):
    sums = jnp.sum(sums_ref[...], axis=0)
    counts = jnp.maximum(jnp.sum(counts_ref[...], axis=0), 1.0)
    fv = sums / counts
    fv = jnp.where(jnp.abs(sums) < 0.01, 0.0, fv)
    gcol = lax.broadcasted_iota(jnp.int32, (BN, G), 1)
    onehot = jnp.where(gid_ref[...] == gcol, 1.0, 0.0)
    corr = lax.dot_general(onehot, fv, (((1,), (1,)), ((), ())),
                           preferred_element_type=jnp.float32)
    out_ref[...] = pm_ref[...] - corr


def _tc_mono(h, nfc, gid, wm, bm):
    gid2 = gid.reshape(N, 1)
    pm1, sums, counts = pl.pallas_call(
        _mono_a_body,
        grid=(NBLK,),
        in_specs=[
            pl.BlockSpec((BN, F), lambda i: (i, 0)),
            pl.BlockSpec((BN, 1), lambda i: (i, 0)),
            pl.BlockSpec((BN, 1), lambda i: (i, 0)),
            pl.BlockSpec((1, H), lambda i: (0, 0)),
            pl.BlockSpec((1, 1), lambda i: (0, 0)),
        ],
        out_specs=(pl.BlockSpec((BN, 1), lambda i: (i, 0)),
                   pl.BlockSpec((1, 1, G), lambda i: (i, 0, 0)),
                   pl.BlockSpec((1, 1, G), lambda i: (i, 0, 0))),
        out_shape=(jax.ShapeDtypeStruct((N, 1), jnp.float32),
                   jax.ShapeDtypeStruct((NBLK, 1, G), jnp.float32),
                   jax.ShapeDtypeStruct((NBLK, 1, G), jnp.float32)),
    )(h, nfc, gid2, wm.reshape(1, H), bm.reshape(1, 1))
    return pl.pallas_call(
        _mono_b_body,
        grid=(NBLK,),
        in_specs=[
            pl.BlockSpec((BN, 1), lambda i: (i, 0)),
            pl.BlockSpec((BN, 1), lambda i: (i, 0)),
            pl.BlockSpec((NBLK, 1, G), lambda i: (0, 0, 0)),
            pl.BlockSpec((NBLK, 1, G), lambda i: (0, 0, 0)),
        ],
        out_specs=pl.BlockSpec((BN, 1), lambda i: (i, 0)),
        out_shape=jax.ShapeDtypeStruct((N, 1), jnp.float32),
    )(pm1, gid2, sums, counts)


def _colmask(width, cols):
    lane = lax.broadcasted_iota(jnp.int32, (1, width), 1)
    m = jnp.zeros((1, width), jnp.float32)
    for ci in cols:
        m = jnp.where(lane == ci, 1.0, m)
    return m


def _final_body(pm_ref, hd_ref, hq_ref, ho_ref, wd_ref, bd_ref,
                wq_ref, bq_ref, wo_ref, bo_ref, out_ref):
    pd = jnp.dot(hd_ref[...], wd_ref[...],
                 preferred_element_type=jnp.float32) + bd_ref[...]
    pq = jnp.dot(hq_ref[...], wq_ref[...],
                 preferred_element_type=jnp.float32) + bq_ref[...]
    mt = (pq[:, 0:1] + pq[:, 3:4] + pq[:, 5:6]) / 3.0
    pq = pq - mt * _colmask(6, (0, 3, 5))
    po = jnp.dot(ho_ref[...], wo_ref[...],
                 preferred_element_type=jnp.float32) + bo_ref[...]
    mt0 = (po[:, 0:1] + po[:, 3:4] + po[:, 5:6]) / 3.0
    mt1 = (po[:, 6:7] + po[:, 1:2] + po[:, 8:9]) / 3.0
    mt2 = (po[:, 9:10] + po[:, 2:3] + po[:, 7:8]) / 3.0
    po = po - mt0 * _colmask(10, (0, 3, 5)) \
            - mt1 * _colmask(10, (6, 1, 8)) \
            - mt2 * _colmask(10, (9, 2, 7))
    out_ref[...] = jnp.concatenate([pm_ref[...], pd, pq, po], axis=1)


def _tc_final(pm, hd, hq, ho, wd, bd, wq, bq, wo, bo):
    return pl.pallas_call(
        _final_body,
        grid=(NBLK,),
        in_specs=[
            pl.BlockSpec((BN, 1), lambda i: (i, 0)),
            pl.BlockSpec((BN, F), lambda i: (i, 0)),
            pl.BlockSpec((BN, F), lambda i: (i, 0)),
            pl.BlockSpec((BN, F), lambda i: (i, 0)),
            pl.BlockSpec((F, 3), lambda i: (0, 0)),
            pl.BlockSpec((1, 3), lambda i: (0, 0)),
            pl.BlockSpec((F, 6), lambda i: (0, 0)),
            pl.BlockSpec((1, 6), lambda i: (0, 0)),
            pl.BlockSpec((F, 10), lambda i: (0, 0)),
            pl.BlockSpec((1, 10), lambda i: (0, 0)),
        ],
        out_specs=pl.BlockSpec((BN, 20), lambda i: (i, 0)),
        out_shape=jax.ShapeDtypeStruct((N, 20), jnp.float32),
    )(pm, hd, hq, ho, wd, bd.reshape(1, 3), wq, bq.reshape(1, 6),
      wo, bo.reshape(1, 10))


def _branches(nfeats, x16_0, efeats, src2, dst2, zeros_acc,
              We1, be1, We2, be2, Wx, bx, Wh1, bh1, Wh2, bh2):
    NB = 4
    st = []
    for b in range(NB):
        i0 = b * 5
        ps, pd = _tc_pre(nfeats, We1[i0, :F], We1[i0, F:2 * F], be1[i0])
        st.append([nfeats, x16_0, efeats, ps, pd])
    for l in range(5):
        zr = [_sc_gather(st[b][3], st[b][4], st[b][1], src2, dst2)
              for b in range(NB)]
        pays = [_tc_edge(zr[b][0], zr[b][1], st[b][2],
                         We1[i, 2 * F:2 * F + De], We1[i, 2 * F + De],
                         We2[i], be2[i], Wx[i, :, 0], bx[i])
                for b in range(NB) for i in [b * 5 + l]]
        accs = [_sc_scatter(pays[b], dst2, zeros_acc) for b in range(NB)]
        for b in range(NB):
            i = b * 5 + l
            nxt = None if l == 4 else (We1[i + 1, :F], We1[i + 1, F:2 * F],
                                       be1[i + 1])
            outs = _tc_node(st[b][0], st[b][1], accs[b], Wh1[i, :F],
                            Wh1[i, F:], bh1[i], Wh2[i], bh2[i], nxt)
            if l == 4:
                st[b] = [outs[0], outs[1], pays[b], None, None]
            else:
                st[b] = [outs[0], outs[1], pays[b], outs[2], outs[3]]
    return [st[b][0] for b in range(4)]


def kernel(nfeats, coordinates, efeats, edge_index, node_graph_ids,
           We1, be1, We2, be2, Wx, bx, Wh1, bh1, Wh2, bh2,
           Wm, bm, Wd, bd, Wq, bq, Wo, bo):
    src2 = edge_index[0].reshape(NROW, CG)
    dst2 = edge_index[1].reshape(NROW, CG)
    x16_0 = jnp.pad(coordinates, ((0, 0), (0, De - 3)))
    zeros_acc = jnp.zeros((N, PAYW), jnp.float32)
    h_mon, h_dip, h_quad, h_oct = _branches(
        nfeats, x16_0, efeats, src2, dst2, zeros_acc,
        We1, be1, We2, be2, Wx, bx, Wh1, bh1, Wh2, bh2)
    pm = _tc_mono(h_mon, nfeats[:, 0:1], node_graph_ids, Wm[:, 0], bm)
    return _tc_final(pm, h_dip, h_quad, h_oct, Wd, bd, Wq, bq, Wo, bo)
